# Initial kernel scaffold; baseline (speedup 1.0000x reference)
import jax
import jax.numpy as jnp
from jax.experimental import pallas as pl


def kernel(x, weight, e_score_correction_bias):
    raise NotImplementedError("write your pallas kernel here")


# speedup vs baseline: 8.1048x; 8.1048x over previous
import functools

import jax
import jax.numpy as jnp
from jax.experimental import pallas as pl
from jax.experimental.pallas import tpu as pltpu

DIM_ = 2048
NE_ = 64
NK_ = 8
NG_ = 8
GSZ_ = NE_ // NG_
NTG_ = 4
SCALE_ = 2.5
NT_ = 8192

_TILE = 512
_INTERPRET = False
_NEG = float("-inf")


def _router_body(x_ref, w_ref, b_ref, wout_ref, iout_ref):
    T = x_ref.shape[0]
    x = x_ref[...]
    w = w_ref[...]
    logits = jax.lax.dot_general(
        w, x, (((1,), (1,)), ((), ())), preferred_element_type=jnp.float32
    )
    orig = jax.nn.sigmoid(logits)
    s = orig + b_ref[...]

    s3 = s.reshape(NG_, GSZ_, T)
    m1 = jnp.max(s3, axis=1)
    iota1 = jax.lax.broadcasted_iota(jnp.int32, (NG_, GSZ_, T), 1)
    eq1 = s3 == m1[:, None, :]
    first = jnp.min(jnp.where(eq1, iota1, GSZ_), axis=1)
    m2 = jnp.max(jnp.where(iota1 == first[:, None, :], _NEG, s3), axis=1)
    gsc = m1 + m2

    gt = (gsc[:, None, :] > gsc[None, :, :]).astype(jnp.int32)
    eqg = gsc[:, None, :] == gsc[None, :, :]
    hlt = (jax.lax.broadcasted_iota(jnp.int32, (NG_, NG_, 1), 0)
           < jax.lax.broadcasted_iota(jnp.int32, (NG_, NG_, 1), 1))
    rank = jnp.sum(gt + (eqg & hlt).astype(jnp.int32), axis=0)
    sel = rank < NTG_
    sel64 = jnp.broadcast_to(sel[:, None, :], (NG_, GSZ_, T)).reshape(NE_, T)
    ms = jnp.where(sel64, s, _NEG)

    iota64 = jax.lax.broadcasted_iota(jnp.int32, (NE_, T), 0)
    ws, idxs = [], []
    for _ in range(NK_):
        m = jnp.max(ms, axis=0, keepdims=True)
        idx = jnp.min(jnp.where(ms == m, iota64, NE_), axis=0, keepdims=True)
        onehot = iota64 == idx
        ws.append(jnp.sum(jnp.where(onehot, orig, 0.0), axis=0, keepdims=True))
        idxs.append(idx)
        ms = jnp.where(onehot, _NEG, ms)
    wcat = jnp.concatenate(ws, axis=0)
    icat = jnp.concatenate(idxs, axis=0)
    wsum = jnp.sum(wcat, axis=0, keepdims=True)
    wout_ref[...] = wcat * (SCALE_ / wsum)
    iout_ref[...] = icat


def kernel(x, weight, e_score_correction_bias):
    b2 = e_score_correction_bias.reshape(NE_, 1)
    n_tiles = NT_ // _TILE
    wout, iout = pl.pallas_call(
        _router_body,
        grid=(n_tiles,),
        in_specs=[
            pl.BlockSpec((_TILE, DIM_), lambda i: (i, 0)),
            pl.BlockSpec((NE_, DIM_), lambda i: (0, 0)),
            pl.BlockSpec((NE_, 1), lambda i: (0, 0)),
        ],
        out_specs=[
            pl.BlockSpec((NK_, _TILE), lambda i: (0, i)),
            pl.BlockSpec((NK_, _TILE), lambda i: (0, i)),
        ],
        out_shape=[
            jax.ShapeDtypeStruct((NK_, NT_), jnp.float32),
            jax.ShapeDtypeStruct((NK_, NT_), jnp.int32),
        ],
        compiler_params=pltpu.CompilerParams(
            dimension_semantics=("arbitrary",),
        ),
        interpret=_INTERPRET,
    )(x, weight, b2)
    return wout.T, iout.T
